# baseline (speedup 1.0000x reference)
import functools

import jax
import jax.numpy as jnp
from jax import lax
from jax.experimental import pallas as pl
from jax.experimental.pallas import tpu as pltpu
from jax.experimental.pallas import tpu_sc as plsc

N = 10000
T = 12
D = 256
E = 160000

NPAD = 10240
NTILES = 16
RPT = NPAD // NTILES
CH = 256
CHR = CH // 128
EPT = 10240
NCH = EPT // CH
EPAD = EPT * NTILES


@functools.cache
def _make_segsum(width):
    mesh = plsc.VectorSubcoreMesh(core_axis_name="c", subcore_axis_name="s")

    @functools.partial(
        pl.kernel,
        mesh=mesh,
        out_type=jax.ShapeDtypeStruct((2 * NPAD, width), jnp.float32),
        scratch_types=[
            pltpu.VMEM((CHR, 128), jnp.int32),
            pltpu.VMEM((CHR, 128), jnp.int32),
            pltpu.VMEM((CH, width), jnp.float32),
            pltpu.VMEM_SHARED((NPAD, width), jnp.float32),
            pltpu.SemaphoreType.DMA,
            pltpu.SemaphoreType.DMA,
        ],
    )
    def k(x_hbm, gidx_hbm, sidx_hbm, zeros_hbm, out_hbm, gi_v, si_v, rows_v,
          acc_sh, sem, sem2):
        c = lax.axis_index("c")
        s = lax.axis_index("s")
        r0 = s * RPT
        pltpu.sync_copy(zeros_hbm.at[pl.ds(r0, RPT)], acc_sh.at[pl.ds(r0, RPT)])
        plsc.subcore_barrier()

        def body(i, carry):
            blk = s * NCH + i
            pltpu.sync_copy(gidx_hbm.at[c, blk], gi_v)
            pltpu.sync_copy(sidx_hbm.at[blk], si_v)
            gds = [pltpu.async_copy(x_hbm.at[gi_v.at[j]],
                                    rows_v.at[pl.ds(j * 128, 128)], sem)
                   for j in range(CHR)]
            for d_ in gds:
                d_.wait()
            sds = [pltpu.async_copy(rows_v.at[pl.ds(j * 128, 128)],
                                    acc_sh.at[si_v.at[j]], sem2, add=True)
                   for j in range(CHR)]
            for d_ in sds:
                d_.wait()
            return carry

        lax.fori_loop(0, NCH, body, 0)
        plsc.subcore_barrier()
        pltpu.sync_copy(acc_sh.at[pl.ds(r0, RPT)],
                        out_hbm.at[pl.ds(c * NPAD + r0, RPT)])

    return k


def _segsum(x, gidx, sidx, zeros):
    return _make_segsum(x.shape[1])(x, gidx, sidx, zeros)


CAPT = 12032
NCHB = CAPT // CH
DUMP = RPT


@functools.cache
def _make_segsum_bkt():
    mesh = plsc.VectorSubcoreMesh(core_axis_name="c", subcore_axis_name="s")

    @functools.partial(
        pl.kernel,
        mesh=mesh,
        out_type=jax.ShapeDtypeStruct((2 * NPAD, 128), jnp.float32),
        scratch_types=[
            pltpu.VMEM((CHR, 128), jnp.int32),
            pltpu.VMEM((CHR, 128), jnp.int32),
            pltpu.VMEM((CH, 128), jnp.float32),
            pltpu.VMEM_SHARED((NPAD, 128), jnp.float32),
            pltpu.SemaphoreType.DMA,
            pltpu.SemaphoreType.DMA,
        ],
    )
    def k(x_hbm, gidx_hbm, sidx_hbm, zeros_hbm, out_hbm, gi_v, si_v, rows_v,
          acc_sh, sem, sem2):
        c = lax.axis_index("c")
        s = lax.axis_index("s")
        r0 = s * RPT
        pltpu.sync_copy(zeros_hbm.at[pl.ds(r0, RPT)], acc_sh.at[pl.ds(r0, RPT)])

        def body(i, carry):
            pltpu.sync_copy(gidx_hbm.at[c, s, i], gi_v)
            pltpu.sync_copy(sidx_hbm.at[s, i], si_v)
            gds = [pltpu.async_copy(x_hbm.at[gi_v.at[j]],
                                    rows_v.at[pl.ds(j * 128, 128)], sem)
                   for j in range(CHR)]
            for d_ in gds:
                d_.wait()
            sds = [pltpu.async_copy(rows_v.at[pl.ds(j * 128, 128)],
                                    acc_sh.at[si_v.at[j]], sem2, add=True)
                   for j in range(CHR)]
            for d_ in sds:
                d_.wait()
            return carry

        lax.fori_loop(0, NCHB, body, 0)
        pltpu.sync_copy(acc_sh.at[pl.ds(r0, RPT)],
                        out_hbm.at[pl.ds(c * NPAD + r0, RPT)])

    return k


@functools.cache
def _make_deg():
    mesh = plsc.VectorSubcoreMesh(core_axis_name="c", subcore_axis_name="s")

    @functools.partial(
        pl.kernel,
        mesh=mesh,
        out_type=jax.ShapeDtypeStruct((NPAD, 128), jnp.float32),
        scratch_types=[
            pltpu.VMEM((CHR, 128), jnp.int32),
            pltpu.VMEM((CH, 128), jnp.float32),
            pltpu.VMEM_SHARED((NPAD, 128), jnp.float32),
            pltpu.SemaphoreType.DMA,
        ],
    )
    def k(sidx_hbm, ones_hbm, zeros_hbm, out_hbm, si_v, rows_v, acc_sh, sem2):
        c = lax.axis_index("c")
        s = lax.axis_index("s")
        r0 = s * RPT
        pltpu.sync_copy(zeros_hbm.at[pl.ds(r0, RPT)], acc_sh.at[pl.ds(r0, RPT)])
        pltpu.sync_copy(ones_hbm, rows_v)
        plsc.subcore_barrier()

        def body(i, carry):
            blk = s * NCH + i
            pltpu.sync_copy(sidx_hbm.at[blk], si_v)
            sds = [pltpu.async_copy(rows_v.at[pl.ds(j * 128, 128)],
                                    acc_sh.at[si_v.at[j]], sem2, add=True)
                   for j in range(CHR)]
            for d_ in sds:
                d_.wait()
            return carry

        lax.fori_loop(0, NCH, body, 0)
        plsc.subcore_barrier()

        @pl.when(c == 0)
        def _():
            pltpu.sync_copy(acc_sh.at[pl.ds(r0, RPT)],
                            out_hbm.at[pl.ds(r0, RPT)])

    return k


BN = 1000


def _gate_body(xt_r, h_r, a1_r, a2_r, c_r, dis_r, w_r, b_r, wci_r, wcf_r,
               wco_r, hn_r, cn_r, x1_r):
    d = dis_r[...]
    h = h_r[...]
    a1 = jnp.concatenate([a1_r[0], a1_r[1]], axis=1)
    a2 = jnp.concatenate([a2_r[0], a2_r[1]], axis=1)
    tx1 = -d * a1
    tx2 = -2.0 * d * a2 - h
    x_cat = jnp.concatenate([xt_r[...], h, tx1, tx2], axis=1)
    p = jnp.dot(x_cat, w_r[...], preferred_element_type=jnp.float32) + b_r[...]
    c_old = c_r[...]
    gi = jax.nn.sigmoid(p[:, :D] + wci_r[...] * c_old)
    gf = jax.nn.sigmoid(p[:, D:2 * D] + wcf_r[...] * c_old)
    gt = jnp.tanh(p[:, 2 * D:3 * D])
    cn = gf * c_old + gi * gt
    go = jax.nn.sigmoid(p[:, 3 * D:] + wco_r[...] * cn)
    hn = go * jnp.tanh(cn)
    hn_r[...] = hn
    cn_r[...] = cn
    x1 = d * hn
    x1_r[0] = x1[:, :128]
    x1_r[1] = x1[:, 128:]


def _gate_step(xt, h, a1, a2, c, dis, wcat, bcat, wci, wcf, wco):
    nb = N // BN
    row = lambda i: (i, 0)
    half = lambda i: (0, i, 0)
    return pl.pallas_call(
        _gate_body,
        grid=(nb,),
        in_specs=[
            pl.BlockSpec((BN, D), row),
            pl.BlockSpec((BN, D), row),
            pl.BlockSpec((2, BN, 128), half),
            pl.BlockSpec((2, BN, 128), half),
            pl.BlockSpec((BN, D), row),
            pl.BlockSpec((BN, 1), row),
            pl.BlockSpec((4 * D, 4 * D), lambda i: (0, 0)),
            pl.BlockSpec((1, 4 * D), lambda i: (0, 0)),
            pl.BlockSpec((1, D), lambda i: (0, 0)),
            pl.BlockSpec((1, D), lambda i: (0, 0)),
            pl.BlockSpec((1, D), lambda i: (0, 0)),
        ],
        out_specs=[
            pl.BlockSpec((BN, D), row),
            pl.BlockSpec((BN, D), row),
            pl.BlockSpec((2, BN, 128), half),
        ],
        out_shape=[
            jax.ShapeDtypeStruct((N, D), jnp.float32),
            jax.ShapeDtypeStruct((N, D), jnp.float32),
            jax.ShapeDtypeStruct((2, NPAD, 128), jnp.float32),
        ],
        compiler_params=pltpu.CompilerParams(
            dimension_semantics=("parallel",)),
    )(xt, h, a1, a2, c, dis, wcat, bcat, wci, wcf, wco)


def _scale_body(a_r, d2_r, o_r):
    o_r[...] = d2_r[...][None] * a_r[...]


def _scale_x2(a1, dis2n):
    return pl.pallas_call(
        _scale_body,
        grid=(N // BN,),
        in_specs=[
            pl.BlockSpec((2, BN, 128), lambda i: (0, i, 0)),
            pl.BlockSpec((BN, 1), lambda i: (i, 0)),
        ],
        out_specs=pl.BlockSpec((2, BN, 128), lambda i: (0, i, 0)),
        out_shape=jax.ShapeDtypeStruct((2, NPAD, 128), jnp.float32),
        compiler_params=pltpu.CompilerParams(
            dimension_semantics=("parallel",)),
    )(a1, dis2n)


def kernel(inputs, edges, W_i, W_f, W_c, W_o, th_i, th_f, th_c, th_o,
           bc_i, bc_f, bc_c, bc_o, b_i, b_f, b_c, b_o, wc_i, wc_f, wc_o):
    src = edges[0].astype(jnp.int32)
    dst = edges[1].astype(jnp.int32)

    padv = jnp.full((EPAD - E,), N, dtype=jnp.int32)
    nblk = NTILES * NCH
    srcp = jnp.concatenate([src, padv]).reshape(nblk, CHR, 128)

    order = jnp.argsort(dst)
    ds_ = dst[order]
    ss_ = src[order]
    bucket = ds_ // RPT
    starts = jnp.searchsorted(ds_, jnp.arange(NTILES, dtype=jnp.int32) * RPT)
    pos = jnp.arange(E, dtype=jnp.int32) - starts[bucket].astype(jnp.int32)
    glist = jnp.zeros((NTILES, CAPT), jnp.int32).at[bucket, pos].set(ss_)
    slist = jnp.full((NTILES, CAPT), N, jnp.int32).at[bucket, pos].set(ds_)
    glist = glist.reshape(NTILES, NCHB, CHR, 128)
    g3mv = jnp.stack([glist, glist + NPAD])
    slist = slist.reshape(NTILES, NCHB, CHR, 128)

    zeros128 = jnp.zeros((NPAD, 128), jnp.float32)
    ones_blk = jnp.ones((CH, 128), jnp.float32)
    assert EPAD == nblk * CH

    deg_out = _make_deg()(srcp, ones_blk, zeros128)
    deg = deg_out[:N, 0]
    dis = jnp.where(deg > 0, 1.0 / jnp.sqrt(jnp.where(deg > 0, deg, 1.0)), 0.0)
    dis_c = dis[:, None]
    dis2n = -(dis_c * dis_c)

    def gcol(w, th):
        return jnp.concatenate([w, th[0], th[1], th[2]], axis=0)

    wcat = jnp.concatenate(
        [gcol(W_i, th_i), gcol(W_f, th_f), gcol(W_c, th_c), gcol(W_o, th_o)],
        axis=1)
    bcat = jnp.concatenate(
        [b_i + bc_i[None, :], b_f + bc_f[None, :], b_c + bc_c[None, :],
         b_o + bc_o[None, :]], axis=1)

    xs = jnp.transpose(inputs, (1, 0, 2))

    h = jnp.zeros((N, D), jnp.float32)
    c = jnp.zeros((N, D), jnp.float32)
    azero = jnp.zeros((2, NPAD, 128), jnp.float32)

    hs = []
    x1 = None
    for t in range(T):
        if t == 0:
            a1 = azero
            a2 = azero
        else:
            mv = _make_segsum_bkt()
            a1 = mv(x1.reshape(2 * NPAD, 128), g3mv, slist,
                    zeros128).reshape(2, NPAD, 128)
            x2 = _scale_x2(a1, dis2n)
            a2 = mv(x2.reshape(2 * NPAD, 128), g3mv, slist,
                    zeros128).reshape(2, NPAD, 128)
        h, c, x1 = _gate_step(xs[t], h, a1, a2, c, dis_c, wcat, bcat,
                              wc_i, wc_f, wc_o)
        hs.append(h)

    series = jnp.stack(hs, axis=1)
    return (series, h, c)


# speedup vs baseline: 3.3957x; 3.3957x over previous
import functools

import jax
import jax.numpy as jnp
from jax import lax
from jax.experimental import pallas as pl
from jax.experimental.pallas import tpu as pltpu
from jax.experimental.pallas import tpu_sc as plsc

N = 10000
T = 12
D = 256
E = 160000

NPAD = 10240
NTILES = 16
RPT = NPAD // NTILES
CH = 256
CHR = CH // 128
EPT = 10240
NCH = EPT // CH
EPAD = EPT * NTILES


@functools.cache
def _make_segsum(width):
    mesh = plsc.VectorSubcoreMesh(core_axis_name="c", subcore_axis_name="s")

    @functools.partial(
        pl.kernel,
        mesh=mesh,
        out_type=jax.ShapeDtypeStruct((2 * NPAD, width), jnp.float32),
        scratch_types=[
            pltpu.VMEM((CHR, 128), jnp.int32),
            pltpu.VMEM((CHR, 128), jnp.int32),
            pltpu.VMEM((CH, width), jnp.float32),
            pltpu.VMEM_SHARED((NPAD, width), jnp.float32),
            pltpu.SemaphoreType.DMA,
            pltpu.SemaphoreType.DMA,
        ],
    )
    def k(x_hbm, gidx_hbm, sidx_hbm, zeros_hbm, out_hbm, gi_v, si_v, rows_v,
          acc_sh, sem, sem2):
        c = lax.axis_index("c")
        s = lax.axis_index("s")
        r0 = s * RPT
        pltpu.sync_copy(zeros_hbm.at[pl.ds(r0, RPT)], acc_sh.at[pl.ds(r0, RPT)])
        plsc.subcore_barrier()

        def body(i, carry):
            blk = s * NCH + i
            pltpu.sync_copy(gidx_hbm.at[c, blk], gi_v)
            pltpu.sync_copy(sidx_hbm.at[blk], si_v)
            gds = [pltpu.async_copy(x_hbm.at[gi_v.at[j]],
                                    rows_v.at[pl.ds(j * 128, 128)], sem)
                   for j in range(CHR)]
            for d_ in gds:
                d_.wait()
            sds = [pltpu.async_copy(rows_v.at[pl.ds(j * 128, 128)],
                                    acc_sh.at[si_v.at[j]], sem2, add=True)
                   for j in range(CHR)]
            for d_ in sds:
                d_.wait()
            return carry

        lax.fori_loop(0, NCH, body, 0)
        plsc.subcore_barrier()
        pltpu.sync_copy(acc_sh.at[pl.ds(r0, RPT)],
                        out_hbm.at[pl.ds(c * NPAD + r0, RPT)])

    return k


def _segsum(x, gidx, sidx, zeros):
    return _make_segsum(x.shape[1])(x, gidx, sidx, zeros)


CAPT = 12032
NCHB = CAPT // CH
DUMP = RPT


@functools.cache
def _make_segsum_bkt():
    mesh = plsc.VectorSubcoreMesh(core_axis_name="c", subcore_axis_name="s")

    @functools.partial(
        pl.kernel,
        mesh=mesh,
        out_type=jax.ShapeDtypeStruct((2 * NPAD, 128), jnp.float32),
        scratch_types=[
            pltpu.VMEM((CHR, 128), jnp.int32),
            pltpu.VMEM((CHR, 128), jnp.int32),
            pltpu.VMEM((CH, 128), jnp.float32),
            pltpu.VMEM_SHARED((NPAD, 128), jnp.float32),
            pltpu.SemaphoreType.DMA,
            pltpu.SemaphoreType.DMA,
        ],
    )
    def k(x_hbm, gidx_hbm, sidx_hbm, zeros_hbm, out_hbm, gi_v, si_v, rows_v,
          acc_sh, sem, sem2):
        c = lax.axis_index("c")
        s = lax.axis_index("s")
        r0 = s * RPT
        pltpu.sync_copy(zeros_hbm.at[pl.ds(r0, RPT)], acc_sh.at[pl.ds(r0, RPT)])

        def body(i, carry):
            pltpu.sync_copy(gidx_hbm.at[c, s, i], gi_v)
            pltpu.sync_copy(sidx_hbm.at[s, i], si_v)
            gds = [pltpu.async_copy(x_hbm.at[gi_v.at[j]],
                                    rows_v.at[pl.ds(j * 128, 128)], sem)
                   for j in range(CHR)]
            for d_ in gds:
                d_.wait()
            sds = [pltpu.async_copy(rows_v.at[pl.ds(j * 128, 128)],
                                    acc_sh.at[si_v.at[j]], sem2, add=True)
                   for j in range(CHR)]
            for d_ in sds:
                d_.wait()
            return carry

        lax.fori_loop(0, NCHB, body, 0)
        pltpu.sync_copy(acc_sh.at[pl.ds(r0, RPT)],
                        out_hbm.at[pl.ds(c * NPAD + r0, RPT)])

    return k


@functools.cache
def _make_deg():
    mesh = plsc.VectorSubcoreMesh(core_axis_name="c", subcore_axis_name="s")

    @functools.partial(
        pl.kernel,
        mesh=mesh,
        out_type=jax.ShapeDtypeStruct((NPAD, 128), jnp.float32),
        scratch_types=[
            pltpu.VMEM((CHR, 128), jnp.int32),
            pltpu.VMEM((CH, 128), jnp.float32),
            pltpu.VMEM_SHARED((NPAD, 128), jnp.float32),
            pltpu.SemaphoreType.DMA,
        ],
    )
    def k(sidx_hbm, ones_hbm, zeros_hbm, out_hbm, si_v, rows_v, acc_sh, sem2):
        c = lax.axis_index("c")
        s = lax.axis_index("s")
        r0 = s * RPT
        pltpu.sync_copy(zeros_hbm.at[pl.ds(r0, RPT)], acc_sh.at[pl.ds(r0, RPT)])
        pltpu.sync_copy(ones_hbm, rows_v)
        plsc.subcore_barrier()

        def body(i, carry):
            blk = s * NCH + i
            pltpu.sync_copy(sidx_hbm.at[blk], si_v)
            sds = [pltpu.async_copy(rows_v.at[pl.ds(j * 128, 128)],
                                    acc_sh.at[si_v.at[j]], sem2, add=True)
                   for j in range(CHR)]
            for d_ in sds:
                d_.wait()
            return carry

        lax.fori_loop(0, NCH, body, 0)
        plsc.subcore_barrier()

        @pl.when(c == 0)
        def _():
            pltpu.sync_copy(acc_sh.at[pl.ds(r0, RPT)],
                            out_hbm.at[pl.ds(r0, RPT)])

    return k


BN = 1000


def _gate_body(xt_r, h_r, a1_r, a2_r, c_r, dis_r, w_r, b_r, wci_r, wcf_r,
               wco_r, hn_r, cn_r, x1_r):
    d = dis_r[...]
    h = h_r[...]
    a1 = jnp.concatenate([a1_r[0], a1_r[1]], axis=1)
    a2 = jnp.concatenate([a2_r[0], a2_r[1]], axis=1)
    tx1 = -d * a1
    tx2 = -2.0 * d * a2 - h
    x_cat = jnp.concatenate([xt_r[...], h, tx1, tx2], axis=1)
    p = jnp.dot(x_cat, w_r[...], preferred_element_type=jnp.float32) + b_r[...]
    c_old = c_r[...]
    gi = jax.nn.sigmoid(p[:, :D] + wci_r[...] * c_old)
    gf = jax.nn.sigmoid(p[:, D:2 * D] + wcf_r[...] * c_old)
    gt = jnp.tanh(p[:, 2 * D:3 * D])
    cn = gf * c_old + gi * gt
    go = jax.nn.sigmoid(p[:, 3 * D:] + wco_r[...] * cn)
    hn = go * jnp.tanh(cn)
    hn_r[...] = hn
    cn_r[...] = cn
    x1 = d * hn
    x1_r[0] = x1[:, :128]
    x1_r[1] = x1[:, 128:]


def _gate_step(xt, h, a1, a2, c, dis, wcat, bcat, wci, wcf, wco):
    nb = N // BN
    row = lambda i: (i, 0)
    half = lambda i: (0, i, 0)
    return pl.pallas_call(
        _gate_body,
        grid=(nb,),
        in_specs=[
            pl.BlockSpec((BN, D), row),
            pl.BlockSpec((BN, D), row),
            pl.BlockSpec((2, BN, 128), half),
            pl.BlockSpec((2, BN, 128), half),
            pl.BlockSpec((BN, D), row),
            pl.BlockSpec((BN, 1), row),
            pl.BlockSpec((4 * D, 4 * D), lambda i: (0, 0)),
            pl.BlockSpec((1, 4 * D), lambda i: (0, 0)),
            pl.BlockSpec((1, D), lambda i: (0, 0)),
            pl.BlockSpec((1, D), lambda i: (0, 0)),
            pl.BlockSpec((1, D), lambda i: (0, 0)),
        ],
        out_specs=[
            pl.BlockSpec((BN, D), row),
            pl.BlockSpec((BN, D), row),
            pl.BlockSpec((2, BN, 128), half),
        ],
        out_shape=[
            jax.ShapeDtypeStruct((N, D), jnp.float32),
            jax.ShapeDtypeStruct((N, D), jnp.float32),
            jax.ShapeDtypeStruct((2, NPAD, 128), jnp.float32),
        ],
        compiler_params=pltpu.CompilerParams(
            dimension_semantics=("parallel",)),
    )(xt, h, a1, a2, c, dis, wcat, bcat, wci, wcf, wco)


def _scale_body(a_r, d2_r, o_r):
    o_r[...] = d2_r[...][None] * a_r[...]


def _scale_x2(a1, dis2n):
    return pl.pallas_call(
        _scale_body,
        grid=(N // BN,),
        in_specs=[
            pl.BlockSpec((2, BN, 128), lambda i: (0, i, 0)),
            pl.BlockSpec((BN, 1), lambda i: (i, 0)),
        ],
        out_specs=pl.BlockSpec((2, BN, 128), lambda i: (0, i, 0)),
        out_shape=jax.ShapeDtypeStruct((2, NPAD, 128), jnp.float32),
        compiler_params=pltpu.CompilerParams(
            dimension_semantics=("parallel",)),
    )(a1, dis2n)


def kernel(inputs, edges, W_i, W_f, W_c, W_o, th_i, th_f, th_c, th_o,
           bc_i, bc_f, bc_c, bc_o, b_i, b_f, b_c, b_o, wc_i, wc_f, wc_o):
    src = edges[0].astype(jnp.int32)
    dst = edges[1].astype(jnp.int32)

    padv = jnp.full((EPAD - E,), N, dtype=jnp.int32)
    nblk = NTILES * NCH
    srcp = jnp.concatenate([src, padv]).reshape(nblk, CHR, 128)

    order = jnp.argsort(src)
    ss_ = src[order]
    ds_ = dst[order]
    src0 = jnp.concatenate([ss_, jnp.zeros((EPAD - E,), jnp.int32)])
    dstp = jnp.concatenate([ds_, padv]).reshape(nblk, CHR, 128)
    g2mv = jnp.stack([src0, src0 + NPAD]).reshape(2, nblk, CHR, 128)

    zeros128 = jnp.zeros((NPAD, 128), jnp.float32)
    ones_blk = jnp.ones((CH, 128), jnp.float32)
    assert EPAD == nblk * CH

    deg_out = _make_deg()(srcp, ones_blk, zeros128)
    deg = deg_out[:N, 0]
    dis = jnp.where(deg > 0, 1.0 / jnp.sqrt(jnp.where(deg > 0, deg, 1.0)), 0.0)
    dis_c = dis[:, None]
    dis2n = -(dis_c * dis_c)

    def gcol(w, th):
        return jnp.concatenate([w, th[0], th[1], th[2]], axis=0)

    wcat = jnp.concatenate(
        [gcol(W_i, th_i), gcol(W_f, th_f), gcol(W_c, th_c), gcol(W_o, th_o)],
        axis=1)
    bcat = jnp.concatenate(
        [b_i + bc_i[None, :], b_f + bc_f[None, :], b_c + bc_c[None, :],
         b_o + bc_o[None, :]], axis=1)

    xs = jnp.transpose(inputs, (1, 0, 2))

    h = jnp.zeros((N, D), jnp.float32)
    c = jnp.zeros((N, D), jnp.float32)
    azero = jnp.zeros((2, NPAD, 128), jnp.float32)

    hs = []
    x1 = None
    for t in range(T):
        if t == 0:
            a1 = azero
            a2 = azero
        else:
            a1 = _segsum(x1.reshape(2 * NPAD, 128), g2mv, dstp,
                         zeros128).reshape(2, NPAD, 128)
            x2 = _scale_x2(a1, dis2n)
            a2 = _segsum(x2.reshape(2 * NPAD, 128), g2mv, dstp,
                         zeros128).reshape(2, NPAD, 128)
        h, c, x1 = _gate_step(xs[t], h, a1, a2, c, dis_c, wcat, bcat,
                              wc_i, wc_f, wc_o)
        hs.append(h)

    series = jnp.stack(hs, axis=1)
    return (series, h, c)


# speedup vs baseline: 3.8307x; 1.1281x over previous
import functools

import jax
import jax.numpy as jnp
from jax import lax
from jax.experimental import pallas as pl
from jax.experimental.pallas import tpu as pltpu
from jax.experimental.pallas import tpu_sc as plsc

N = 10000
T = 12
D = 256
E = 160000

NPAD = 10240
NTILES = 16
RPT = NPAD // NTILES
CH = 256
CHR = CH // 128
EPT = 10240
NCH = EPT // CH
EPAD = EPT * NTILES


@functools.cache
def _make_segsum(width):
    mesh = plsc.VectorSubcoreMesh(core_axis_name="c", subcore_axis_name="s")

    @functools.partial(
        pl.kernel,
        mesh=mesh,
        out_type=jax.ShapeDtypeStruct((2 * NPAD, width), jnp.float32),
        scratch_types=[
            pltpu.VMEM((CH,), jnp.int32),
            pltpu.VMEM((CH,), jnp.int32),
            pltpu.VMEM((CH, width), jnp.float32),
            pltpu.VMEM_SHARED((NPAD, width), jnp.float32),
            pltpu.SemaphoreType.DMA,
            pltpu.SemaphoreType.DMA,
        ],
    )
    def k(x_hbm, gidx_hbm, sidx_hbm, zeros_hbm, out_hbm, gi_v, si_v, rows_v,
          acc_sh, sem, sem2):
        c = lax.axis_index("c")
        s = lax.axis_index("s")
        r0 = s * RPT
        pltpu.sync_copy(zeros_hbm.at[pl.ds(r0, RPT)], acc_sh.at[pl.ds(r0, RPT)])
        plsc.subcore_barrier()

        def body(i, carry):
            blk = s * NCH + i
            pltpu.sync_copy(gidx_hbm.at[c, blk], gi_v)
            pltpu.sync_copy(sidx_hbm.at[blk], si_v)
            pltpu.async_copy(x_hbm.at[gi_v], rows_v, sem).wait()
            pltpu.async_copy(rows_v, acc_sh.at[si_v], sem2, add=True).wait()
            return carry

        lax.fori_loop(0, NCH, body, 0)
        plsc.subcore_barrier()
        pltpu.sync_copy(acc_sh.at[pl.ds(r0, RPT)],
                        out_hbm.at[pl.ds(c * NPAD + r0, RPT)])

    return k


def _segsum(x, gidx, sidx, zeros):
    return _make_segsum(x.shape[1])(x, gidx, sidx, zeros)


@functools.cache
def _make_deg():
    mesh = plsc.VectorSubcoreMesh(core_axis_name="c", subcore_axis_name="s")

    @functools.partial(
        pl.kernel,
        mesh=mesh,
        out_type=jax.ShapeDtypeStruct((NPAD, 128), jnp.float32),
        scratch_types=[
            pltpu.VMEM((CH,), jnp.int32),
            pltpu.VMEM((CH, 128), jnp.float32),
            pltpu.VMEM_SHARED((NPAD, 128), jnp.float32),
            pltpu.SemaphoreType.DMA,
        ],
    )
    def k(sidx_hbm, ones_hbm, zeros_hbm, out_hbm, si_v, rows_v, acc_sh, sem2):
        c = lax.axis_index("c")
        s = lax.axis_index("s")
        r0 = s * RPT
        pltpu.sync_copy(zeros_hbm.at[pl.ds(r0, RPT)], acc_sh.at[pl.ds(r0, RPT)])
        pltpu.sync_copy(ones_hbm, rows_v)
        plsc.subcore_barrier()

        def body(i, carry):
            blk = s * NCH + i
            pltpu.sync_copy(sidx_hbm.at[blk], si_v)
            pltpu.async_copy(rows_v, acc_sh.at[si_v], sem2, add=True).wait()
            return carry

        lax.fori_loop(0, NCH, body, 0)
        plsc.subcore_barrier()

        @pl.when(c == 0)
        def _():
            pltpu.sync_copy(acc_sh.at[pl.ds(r0, RPT)],
                            out_hbm.at[pl.ds(r0, RPT)])

    return k


BN = 1000


def _gate_body(xt_r, h_r, a1_r, a2_r, c_r, dis_r, w_r, b_r, wci_r, wcf_r,
               wco_r, hn_r, cn_r, x1_r):
    d = dis_r[...]
    h = h_r[...]
    a1 = jnp.concatenate([a1_r[0], a1_r[1]], axis=1)
    a2 = jnp.concatenate([a2_r[0], a2_r[1]], axis=1)
    tx1 = -d * a1
    tx2 = -2.0 * d * a2 - h
    x_cat = jnp.concatenate([xt_r[...], h, tx1, tx2], axis=1)
    p = jnp.dot(x_cat, w_r[...], preferred_element_type=jnp.float32) + b_r[...]
    c_old = c_r[...]
    gi = jax.nn.sigmoid(p[:, :D] + wci_r[...] * c_old)
    gf = jax.nn.sigmoid(p[:, D:2 * D] + wcf_r[...] * c_old)
    gt = jnp.tanh(p[:, 2 * D:3 * D])
    cn = gf * c_old + gi * gt
    go = jax.nn.sigmoid(p[:, 3 * D:] + wco_r[...] * cn)
    hn = go * jnp.tanh(cn)
    hn_r[...] = hn
    cn_r[...] = cn
    x1 = d * hn
    x1_r[0] = x1[:, :128]
    x1_r[1] = x1[:, 128:]


def _gate_step(xt, h, a1, a2, c, dis, wcat, bcat, wci, wcf, wco):
    nb = N // BN
    row = lambda i: (i, 0)
    half = lambda i: (0, i, 0)
    return pl.pallas_call(
        _gate_body,
        grid=(nb,),
        in_specs=[
            pl.BlockSpec((BN, D), row),
            pl.BlockSpec((BN, D), row),
            pl.BlockSpec((2, BN, 128), half),
            pl.BlockSpec((2, BN, 128), half),
            pl.BlockSpec((BN, D), row),
            pl.BlockSpec((BN, 1), row),
            pl.BlockSpec((4 * D, 4 * D), lambda i: (0, 0)),
            pl.BlockSpec((1, 4 * D), lambda i: (0, 0)),
            pl.BlockSpec((1, D), lambda i: (0, 0)),
            pl.BlockSpec((1, D), lambda i: (0, 0)),
            pl.BlockSpec((1, D), lambda i: (0, 0)),
        ],
        out_specs=[
            pl.BlockSpec((BN, D), row),
            pl.BlockSpec((BN, D), row),
            pl.BlockSpec((2, BN, 128), half),
        ],
        out_shape=[
            jax.ShapeDtypeStruct((N, D), jnp.float32),
            jax.ShapeDtypeStruct((N, D), jnp.float32),
            jax.ShapeDtypeStruct((2, NPAD, 128), jnp.float32),
        ],
        compiler_params=pltpu.CompilerParams(
            dimension_semantics=("parallel",)),
    )(xt, h, a1, a2, c, dis, wcat, bcat, wci, wcf, wco)


def _scale_body(a_r, d2_r, o_r):
    o_r[...] = d2_r[...][None] * a_r[...]


def _scale_x2(a1, dis2n):
    return pl.pallas_call(
        _scale_body,
        grid=(N // BN,),
        in_specs=[
            pl.BlockSpec((2, BN, 128), lambda i: (0, i, 0)),
            pl.BlockSpec((BN, 1), lambda i: (i, 0)),
        ],
        out_specs=pl.BlockSpec((2, BN, 128), lambda i: (0, i, 0)),
        out_shape=jax.ShapeDtypeStruct((2, NPAD, 128), jnp.float32),
        compiler_params=pltpu.CompilerParams(
            dimension_semantics=("parallel",)),
    )(a1, dis2n)


def kernel(inputs, edges, W_i, W_f, W_c, W_o, th_i, th_f, th_c, th_o,
           bc_i, bc_f, bc_c, bc_o, b_i, b_f, b_c, b_o, wc_i, wc_f, wc_o):
    src = edges[0].astype(jnp.int32)
    dst = edges[1].astype(jnp.int32)

    padv = jnp.full((EPAD - E,), N, dtype=jnp.int32)
    nblk = NTILES * NCH
    srcp = jnp.concatenate([src, padv]).reshape(nblk, CH)
    src0 = jnp.concatenate([src, jnp.zeros((EPAD - E,), jnp.int32)])
    dstp = jnp.concatenate([dst, padv]).reshape(nblk, CH)
    g2mv = jnp.stack([src0, src0 + NPAD]).reshape(2, nblk, CH)

    zeros128 = jnp.zeros((NPAD, 128), jnp.float32)
    ones_blk = jnp.ones((CH, 128), jnp.float32)
    assert EPAD == nblk * CH

    deg_out = _make_deg()(srcp, ones_blk, zeros128)
    deg = deg_out[:N, 0]
    dis = jnp.where(deg > 0, 1.0 / jnp.sqrt(jnp.where(deg > 0, deg, 1.0)), 0.0)
    dis_c = dis[:, None]
    dis2n = -(dis_c * dis_c)

    def gcol(w, th):
        return jnp.concatenate([w, th[0], th[1], th[2]], axis=0)

    wcat = jnp.concatenate(
        [gcol(W_i, th_i), gcol(W_f, th_f), gcol(W_c, th_c), gcol(W_o, th_o)],
        axis=1)
    bcat = jnp.concatenate(
        [b_i + bc_i[None, :], b_f + bc_f[None, :], b_c + bc_c[None, :],
         b_o + bc_o[None, :]], axis=1)

    xs = jnp.transpose(inputs, (1, 0, 2))

    h = jnp.zeros((N, D), jnp.float32)
    c = jnp.zeros((N, D), jnp.float32)
    azero = jnp.zeros((2, NPAD, 128), jnp.float32)

    hs = []
    x1 = None
    for t in range(T):
        if t == 0:
            a1 = azero
            a2 = azero
        else:
            a1 = _segsum(x1.reshape(2 * NPAD, 128), g2mv, dstp,
                         zeros128).reshape(2, NPAD, 128)
            x2 = _scale_x2(a1, dis2n)
            a2 = _segsum(x2.reshape(2 * NPAD, 128), g2mv, dstp,
                         zeros128).reshape(2, NPAD, 128)
        h, c, x1 = _gate_step(xs[t], h, a1, a2, c, dis_c, wcat, bcat,
                              wc_i, wc_f, wc_o)
        hs.append(h)

    series = jnp.stack(hs, axis=1)
    return (series, h, c)


# speedup vs baseline: 4.2007x; 1.0966x over previous
import functools

import jax
import jax.numpy as jnp
from jax import lax
from jax.experimental import pallas as pl
from jax.experimental.pallas import tpu as pltpu
from jax.experimental.pallas import tpu_sc as plsc

N = 10000
T = 12
D = 256
E = 160000

NPAD = 10240
NTILES = 16
RPT = NPAD // NTILES
CH = 128
EPT = 10240
NCH = EPT // CH
NPAIR = NCH // 2
EPAD = EPT * NTILES


@functools.cache
def _make_segsum(width):
    mesh = plsc.VectorSubcoreMesh(core_axis_name="c", subcore_axis_name="s")

    @functools.partial(
        pl.kernel,
        mesh=mesh,
        out_type=jax.ShapeDtypeStruct((2 * NPAD, width), jnp.float32),
        scratch_types=[
            pltpu.VMEM((CH,), jnp.int32),
            pltpu.VMEM((CH,), jnp.int32),
            pltpu.VMEM((CH,), jnp.int32),
            pltpu.VMEM((CH,), jnp.int32),
            pltpu.VMEM((CH, width), jnp.float32),
            pltpu.VMEM((CH, width), jnp.float32),
            pltpu.VMEM_SHARED((NPAD, width), jnp.float32),
            pltpu.SemaphoreType.DMA,
            pltpu.SemaphoreType.DMA,
            pltpu.SemaphoreType.DMA,
            pltpu.SemaphoreType.DMA,
        ],
    )
    def k(x_hbm, gidx_hbm, sidx_hbm, zeros_hbm, out_hbm, gia, sia, gib, sib,
          bufa, bufb, acc_sh, gsa, gsb, ssa, ssb):
        c = lax.axis_index("c")
        s = lax.axis_index("s")
        r0 = s * RPT
        pltpu.sync_copy(zeros_hbm.at[pl.ds(r0, RPT)], acc_sh.at[pl.ds(r0, RPT)])
        plsc.subcore_barrier()

        base = s * NCH

        def load(gi, si, blk):
            pltpu.sync_copy(gidx_hbm.at[c, blk], gi)
            pltpu.sync_copy(sidx_hbm.at[blk], si)

        load(gia, sia, base)
        pltpu.async_copy(x_hbm.at[gia], bufa, gsa)
        load(gib, sib, base + 1)
        pltpu.async_copy(x_hbm.at[gib], bufb, gsb)

        def pair(g, carry):
            pltpu.make_async_copy(x_hbm.at[gia], bufa, gsa).wait()
            sa = pltpu.async_copy(bufa, acc_sh.at[sia], ssa, add=True)
            pltpu.make_async_copy(x_hbm.at[gib], bufb, gsb).wait()
            sb = pltpu.async_copy(bufb, acc_sh.at[sib], ssb, add=True)
            sa.wait()
            load(gia, sia, base + 2 * g + 2)
            pltpu.async_copy(x_hbm.at[gia], bufa, gsa)
            sb.wait()
            load(gib, sib, base + 2 * g + 3)
            pltpu.async_copy(x_hbm.at[gib], bufb, gsb)
            return carry

        lax.fori_loop(0, NPAIR - 1, pair, 0)
        pltpu.make_async_copy(x_hbm.at[gia], bufa, gsa).wait()
        pltpu.async_copy(bufa, acc_sh.at[sia], ssa, add=True).wait()
        pltpu.make_async_copy(x_hbm.at[gib], bufb, gsb).wait()
        pltpu.async_copy(bufb, acc_sh.at[sib], ssb, add=True).wait()
        plsc.subcore_barrier()
        pltpu.sync_copy(acc_sh.at[pl.ds(r0, RPT)],
                        out_hbm.at[pl.ds(c * NPAD + r0, RPT)])

    return k


def _segsum(x, gidx, sidx, zeros):
    return _make_segsum(x.shape[1])(x, gidx, sidx, zeros)


@functools.cache
def _make_deg():
    mesh = plsc.VectorSubcoreMesh(core_axis_name="c", subcore_axis_name="s")

    @functools.partial(
        pl.kernel,
        mesh=mesh,
        out_type=jax.ShapeDtypeStruct((NPAD, 128), jnp.float32),
        scratch_types=[
            pltpu.VMEM((CH,), jnp.int32),
            pltpu.VMEM((CH, 128), jnp.float32),
            pltpu.VMEM_SHARED((NPAD, 128), jnp.float32),
            pltpu.SemaphoreType.DMA,
        ],
    )
    def k(sidx_hbm, ones_hbm, zeros_hbm, out_hbm, si_v, rows_v, acc_sh, sem2):
        c = lax.axis_index("c")
        s = lax.axis_index("s")
        r0 = s * RPT
        pltpu.sync_copy(zeros_hbm.at[pl.ds(r0, RPT)], acc_sh.at[pl.ds(r0, RPT)])
        pltpu.sync_copy(ones_hbm, rows_v)
        plsc.subcore_barrier()

        def body(i, carry):
            blk = s * NCH + i
            pltpu.sync_copy(sidx_hbm.at[blk], si_v)
            pltpu.async_copy(rows_v, acc_sh.at[si_v], sem2, add=True).wait()
            return carry

        lax.fori_loop(0, NCH, body, 0)
        plsc.subcore_barrier()

        @pl.when(c == 0)
        def _():
            pltpu.sync_copy(acc_sh.at[pl.ds(r0, RPT)],
                            out_hbm.at[pl.ds(r0, RPT)])

    return k


BN = 1000


def _gate_body(xt_r, h_r, a1_r, a2_r, c_r, dis_r, w_r, b_r, wci_r, wcf_r,
               wco_r, hn_r, cn_r, x1_r):
    d = dis_r[...]
    h = h_r[...]
    a1 = jnp.concatenate([a1_r[0], a1_r[1]], axis=1)
    a2 = jnp.concatenate([a2_r[0], a2_r[1]], axis=1)
    tx1 = -d * a1
    tx2 = -2.0 * d * a2 - h
    x_cat = jnp.concatenate([xt_r[...], h, tx1, tx2], axis=1)
    p = jnp.dot(x_cat, w_r[...], preferred_element_type=jnp.float32) + b_r[...]
    c_old = c_r[...]
    gi = jax.nn.sigmoid(p[:, :D] + wci_r[...] * c_old)
    gf = jax.nn.sigmoid(p[:, D:2 * D] + wcf_r[...] * c_old)
    gt = jnp.tanh(p[:, 2 * D:3 * D])
    cn = gf * c_old + gi * gt
    go = jax.nn.sigmoid(p[:, 3 * D:] + wco_r[...] * cn)
    hn = go * jnp.tanh(cn)
    hn_r[...] = hn
    cn_r[...] = cn
    x1 = d * hn
    x1_r[0] = x1[:, :128]
    x1_r[1] = x1[:, 128:]


def _gate_step(xt, h, a1, a2, c, dis, wcat, bcat, wci, wcf, wco):
    nb = N // BN
    row = lambda i: (i, 0)
    half = lambda i: (0, i, 0)
    return pl.pallas_call(
        _gate_body,
        grid=(nb,),
        in_specs=[
            pl.BlockSpec((BN, D), row),
            pl.BlockSpec((BN, D), row),
            pl.BlockSpec((2, BN, 128), half),
            pl.BlockSpec((2, BN, 128), half),
            pl.BlockSpec((BN, D), row),
            pl.BlockSpec((BN, 1), row),
            pl.BlockSpec((4 * D, 4 * D), lambda i: (0, 0)),
            pl.BlockSpec((1, 4 * D), lambda i: (0, 0)),
            pl.BlockSpec((1, D), lambda i: (0, 0)),
            pl.BlockSpec((1, D), lambda i: (0, 0)),
            pl.BlockSpec((1, D), lambda i: (0, 0)),
        ],
        out_specs=[
            pl.BlockSpec((BN, D), row),
            pl.BlockSpec((BN, D), row),
            pl.BlockSpec((2, BN, 128), half),
        ],
        out_shape=[
            jax.ShapeDtypeStruct((N, D), jnp.float32),
            jax.ShapeDtypeStruct((N, D), jnp.float32),
            jax.ShapeDtypeStruct((2, NPAD, 128), jnp.float32),
        ],
        compiler_params=pltpu.CompilerParams(
            dimension_semantics=("parallel",)),
    )(xt, h, a1, a2, c, dis, wcat, bcat, wci, wcf, wco)


def _scale_body(a_r, d2_r, o_r):
    o_r[...] = d2_r[...][None] * a_r[...]


def _scale_x2(a1, dis2n):
    return pl.pallas_call(
        _scale_body,
        grid=(N // BN,),
        in_specs=[
            pl.BlockSpec((2, BN, 128), lambda i: (0, i, 0)),
            pl.BlockSpec((BN, 1), lambda i: (i, 0)),
        ],
        out_specs=pl.BlockSpec((2, BN, 128), lambda i: (0, i, 0)),
        out_shape=jax.ShapeDtypeStruct((2, NPAD, 128), jnp.float32),
        compiler_params=pltpu.CompilerParams(
            dimension_semantics=("parallel",)),
    )(a1, dis2n)


def kernel(inputs, edges, W_i, W_f, W_c, W_o, th_i, th_f, th_c, th_o,
           bc_i, bc_f, bc_c, bc_o, b_i, b_f, b_c, b_o, wc_i, wc_f, wc_o):
    src = edges[0].astype(jnp.int32)
    dst = edges[1].astype(jnp.int32)

    padv = jnp.full((EPAD - E,), N, dtype=jnp.int32)
    nblk = NTILES * NCH
    srcp = jnp.concatenate([src, padv]).reshape(nblk, CH)
    src0 = jnp.concatenate([src, jnp.zeros((EPAD - E,), jnp.int32)])
    dstp = jnp.concatenate([dst, padv]).reshape(nblk, CH)
    g2mv = jnp.stack([src0, src0 + NPAD]).reshape(2, nblk, CH)

    zeros128 = jnp.zeros((NPAD, 128), jnp.float32)
    ones_blk = jnp.ones((CH, 128), jnp.float32)
    assert EPAD == nblk * CH

    deg_out = _make_deg()(srcp, ones_blk, zeros128)
    deg = deg_out[:N, 0]
    dis = jnp.where(deg > 0, 1.0 / jnp.sqrt(jnp.where(deg > 0, deg, 1.0)), 0.0)
    dis_c = dis[:, None]
    dis2n = -(dis_c * dis_c)

    def gcol(w, th):
        return jnp.concatenate([w, th[0], th[1], th[2]], axis=0)

    wcat = jnp.concatenate(
        [gcol(W_i, th_i), gcol(W_f, th_f), gcol(W_c, th_c), gcol(W_o, th_o)],
        axis=1)
    bcat = jnp.concatenate(
        [b_i + bc_i[None, :], b_f + bc_f[None, :], b_c + bc_c[None, :],
         b_o + bc_o[None, :]], axis=1)

    xs = jnp.transpose(inputs, (1, 0, 2))

    h = jnp.zeros((N, D), jnp.float32)
    c = jnp.zeros((N, D), jnp.float32)
    azero = jnp.zeros((2, NPAD, 128), jnp.float32)

    hs = []
    x1 = None
    for t in range(T):
        if t == 0:
            a1 = azero
            a2 = azero
        else:
            a1 = _segsum(x1.reshape(2 * NPAD, 128), g2mv, dstp,
                         zeros128).reshape(2, NPAD, 128)
            x2 = _scale_x2(a1, dis2n)
            a2 = _segsum(x2.reshape(2 * NPAD, 128), g2mv, dstp,
                         zeros128).reshape(2, NPAD, 128)
        h, c, x1 = _gate_step(xs[t], h, a1, a2, c, dis_c, wcat, bcat,
                              wc_i, wc_f, wc_o)
        hs.append(h)

    series = jnp.stack(hs, axis=1)
    return (series, h, c)


# speedup vs baseline: 4.3734x; 1.0411x over previous
import functools

import jax
import jax.numpy as jnp
from jax import lax
from jax.experimental import pallas as pl
from jax.experimental.pallas import tpu as pltpu
from jax.experimental.pallas import tpu_sc as plsc

N = 10000
T = 12
D = 256
E = 160000

NPAD = 10240
NTILES = 16
RPT = NPAD // NTILES
CH = 128
EPT = 10112
EPAD = EPT * NTILES


@functools.cache
def _make_segsum(width):
    mesh = plsc.VectorSubcoreMesh(core_axis_name="c", subcore_axis_name="s")

    @functools.partial(
        pl.kernel,
        mesh=mesh,
        out_type=jax.ShapeDtypeStruct((2 * NPAD, width), jnp.float32),
        scratch_types=[
            pltpu.VMEM((CH,), jnp.int32),
            pltpu.VMEM((CH,), jnp.int32),
            pltpu.VMEM((CH, width), jnp.float32),
            pltpu.VMEM_SHARED((NPAD, width), jnp.float32),
            pltpu.SemaphoreType.DMA,
        ],
    )
    def k(x_hbm, gidx_hbm, sidx_hbm, zeros_hbm, out_hbm, gi_v, si_v, rows_v,
          acc_sh, sem):
        c = lax.axis_index("c")
        s = lax.axis_index("s")
        r0 = s * RPT
        pltpu.sync_copy(zeros_hbm.at[pl.ds(r0, RPT)], acc_sh.at[pl.ds(r0, RPT)])
        plsc.subcore_barrier()

        base = s * EPT

        def body(i, carry):
            off = pl.multiple_of(base + i * CH, CH)
            pltpu.sync_copy(gidx_hbm.at[c, pl.ds(off, CH)], gi_v)
            pltpu.sync_copy(sidx_hbm.at[pl.ds(off, CH)], si_v)
            pltpu.async_copy(x_hbm.at[gi_v], rows_v, sem).wait()
            pltpu.sync_copy(rows_v, acc_sh.at[si_v], add=True)
            return carry

        lax.fori_loop(0, EPT // CH, body, 0)
        plsc.subcore_barrier()
        pltpu.sync_copy(acc_sh.at[pl.ds(r0, RPT)],
                        out_hbm.at[pl.ds(c * NPAD + r0, RPT)])

    return k


def _segsum(x, gidx, sidx, zeros):
    return _make_segsum(x.shape[1])(x, gidx, sidx, zeros)


@functools.cache
def _make_deg():
    mesh = plsc.VectorSubcoreMesh(core_axis_name="c", subcore_axis_name="s")

    @functools.partial(
        pl.kernel,
        mesh=mesh,
        out_type=jax.ShapeDtypeStruct((NPAD, 128), jnp.float32),
        scratch_types=[
            pltpu.VMEM((CH,), jnp.int32),
            pltpu.VMEM((CH, 128), jnp.float32),
            pltpu.VMEM_SHARED((NPAD, 128), jnp.float32),
        ],
    )
    def k(sidx_hbm, ones_hbm, zeros_hbm, out_hbm, si_v, rows_v, acc_sh):
        c = lax.axis_index("c")
        s = lax.axis_index("s")
        r0 = s * RPT
        pltpu.sync_copy(zeros_hbm.at[pl.ds(r0, RPT)], acc_sh.at[pl.ds(r0, RPT)])
        pltpu.sync_copy(ones_hbm, rows_v)
        plsc.subcore_barrier()

        base = s * EPT

        def body(i, carry):
            off = pl.multiple_of(base + i * CH, CH)
            pltpu.sync_copy(sidx_hbm.at[pl.ds(off, CH)], si_v)
            pltpu.sync_copy(rows_v, acc_sh.at[si_v], add=True)
            return carry

        lax.fori_loop(0, EPT // CH, body, 0)
        plsc.subcore_barrier()

        @pl.when(c == 0)
        def _():
            pltpu.sync_copy(acc_sh.at[pl.ds(r0, RPT)],
                            out_hbm.at[pl.ds(r0, RPT)])

    return k


BN = 1000


def _gate_body(xt_r, h_r, a1_r, a2_r, c_r, dis_r, w_r, b_r, wci_r, wcf_r,
               wco_r, hn_r, cn_r, x1_r):
    d = dis_r[...]
    h = h_r[...]
    a1 = jnp.concatenate([a1_r[0], a1_r[1]], axis=1)
    a2 = jnp.concatenate([a2_r[0], a2_r[1]], axis=1)
    tx1 = -d * a1
    tx2 = -2.0 * d * a2 - h
    x_cat = jnp.concatenate([xt_r[...], h, tx1, tx2], axis=1)
    p = jnp.dot(x_cat, w_r[...], preferred_element_type=jnp.float32) + b_r[...]
    c_old = c_r[...]
    gi = jax.nn.sigmoid(p[:, :D] + wci_r[...] * c_old)
    gf = jax.nn.sigmoid(p[:, D:2 * D] + wcf_r[...] * c_old)
    gt = jnp.tanh(p[:, 2 * D:3 * D])
    cn = gf * c_old + gi * gt
    go = jax.nn.sigmoid(p[:, 3 * D:] + wco_r[...] * cn)
    hn = go * jnp.tanh(cn)
    hn_r[...] = hn
    cn_r[...] = cn
    x1 = d * hn
    x1_r[0] = x1[:, :128]
    x1_r[1] = x1[:, 128:]


def _gate_step(xt, h, a1, a2, c, dis, wcat, bcat, wci, wcf, wco):
    nb = N // BN
    row = lambda i: (i, 0)
    half = lambda i: (0, i, 0)
    return pl.pallas_call(
        _gate_body,
        grid=(nb,),
        in_specs=[
            pl.BlockSpec((BN, D), row),
            pl.BlockSpec((BN, D), row),
            pl.BlockSpec((2, BN, 128), half),
            pl.BlockSpec((2, BN, 128), half),
            pl.BlockSpec((BN, D), row),
            pl.BlockSpec((BN, 1), row),
            pl.BlockSpec((4 * D, 4 * D), lambda i: (0, 0)),
            pl.BlockSpec((1, 4 * D), lambda i: (0, 0)),
            pl.BlockSpec((1, D), lambda i: (0, 0)),
            pl.BlockSpec((1, D), lambda i: (0, 0)),
            pl.BlockSpec((1, D), lambda i: (0, 0)),
        ],
        out_specs=[
            pl.BlockSpec((BN, D), row),
            pl.BlockSpec((BN, D), row),
            pl.BlockSpec((2, BN, 128), half),
        ],
        out_shape=[
            jax.ShapeDtypeStruct((N, D), jnp.float32),
            jax.ShapeDtypeStruct((N, D), jnp.float32),
            jax.ShapeDtypeStruct((2, NPAD, 128), jnp.float32),
        ],
        compiler_params=pltpu.CompilerParams(
            dimension_semantics=("parallel",)),
    )(xt, h, a1, a2, c, dis, wcat, bcat, wci, wcf, wco)


def _scale_body(a_r, d2_r, o_r):
    o_r[...] = d2_r[...][None] * a_r[...]


def _scale_x2(a1, dis2n):
    return pl.pallas_call(
        _scale_body,
        grid=(N // BN,),
        in_specs=[
            pl.BlockSpec((2, BN, 128), lambda i: (0, i, 0)),
            pl.BlockSpec((BN, 1), lambda i: (i, 0)),
        ],
        out_specs=pl.BlockSpec((2, BN, 128), lambda i: (0, i, 0)),
        out_shape=jax.ShapeDtypeStruct((2, NPAD, 128), jnp.float32),
        compiler_params=pltpu.CompilerParams(
            dimension_semantics=("parallel",)),
    )(a1, dis2n)


def kernel(inputs, edges, W_i, W_f, W_c, W_o, th_i, th_f, th_c, th_o,
           bc_i, bc_f, bc_c, bc_o, b_i, b_f, b_c, b_o, wc_i, wc_f, wc_o):
    src = edges[0].astype(jnp.int32)
    dst = edges[1].astype(jnp.int32)

    padv = jnp.full((EPAD - E,), N, dtype=jnp.int32)
    pad0 = jnp.zeros((EPAD - E,), dtype=jnp.int32)
    srcp = jnp.concatenate([src, padv])
    src0 = jnp.concatenate([src, pad0])
    dstp = jnp.concatenate([dst, padv])
    g2mv = jnp.stack([src0, src0 + NPAD])

    zeros128 = jnp.zeros((NPAD, 128), jnp.float32)
    ones_blk = jnp.ones((CH, 128), jnp.float32)

    deg_out = _make_deg()(srcp, ones_blk, zeros128)
    deg = deg_out[:N, 0]
    dis = jnp.where(deg > 0, 1.0 / jnp.sqrt(jnp.where(deg > 0, deg, 1.0)), 0.0)
    dis_c = dis[:, None]
    dis2n = -(dis_c * dis_c)

    def gcol(w, th):
        return jnp.concatenate([w, th[0], th[1], th[2]], axis=0)

    wcat = jnp.concatenate(
        [gcol(W_i, th_i), gcol(W_f, th_f), gcol(W_c, th_c), gcol(W_o, th_o)],
        axis=1)
    bcat = jnp.concatenate(
        [b_i + bc_i[None, :], b_f + bc_f[None, :], b_c + bc_c[None, :],
         b_o + bc_o[None, :]], axis=1)

    xs = jnp.transpose(inputs, (1, 0, 2))

    h = jnp.zeros((N, D), jnp.float32)
    c = jnp.zeros((N, D), jnp.float32)
    azero = jnp.zeros((2, NPAD, 128), jnp.float32)

    hs = []
    x1 = None
    for t in range(T):
        if t == 0:
            a1 = azero
            a2 = azero
        else:
            a1 = _segsum(x1.reshape(2 * NPAD, 128), g2mv, dstp,
                         zeros128).reshape(2, NPAD, 128)
            x2 = _scale_x2(a1, dis2n)
            a2 = _segsum(x2.reshape(2 * NPAD, 128), g2mv, dstp,
                         zeros128).reshape(2, NPAD, 128)
        h, c, x1 = _gate_step(xs[t], h, a1, a2, c, dis_c, wcat, bcat,
                              wc_i, wc_f, wc_o)
        hs.append(h)

    series = jnp.stack(hs, axis=1)
    return (series, h, c)
